# baseline (speedup 1.0000x reference)
import jax
import jax.numpy as jnp
from jax.experimental import pallas as pl
from jax.experimental.pallas import tpu as pltpu

_BM = 400


def _gcn_kernel(seq_ref, adj_ref, w_ref, bias_ref, a_ref, out_ref, fts_ref):
    @pl.when(pl.program_id(0) == 0)
    def _():
        fts_ref[...] = jax.lax.dot_general(
            seq_ref[...], w_ref[...],
            dimension_numbers=(((1,), (1,)), ((), ())),
            preferred_element_type=jnp.float32).astype(jnp.bfloat16)

    out = jax.lax.dot_general(
        adj_ref[...].astype(jnp.bfloat16), fts_ref[...],
        dimension_numbers=(((1,), (0,)), ((), ())),
        preferred_element_type=jnp.float32)
    out = out + bias_ref[...]
    a = a_ref[0, 0]
    out_ref[...] = jnp.where(out >= 0, out, a * out)


def kernel(seq, adj, W, bias, prelu_a):
    n, _ = seq.shape
    out_ft = W.shape[0]
    bm = _BM if n % _BM == 0 else min(_BM, n)
    grid = (pl.cdiv(n, bm),)
    return pl.pallas_call(
        _gcn_kernel,
        grid=grid,
        in_specs=[
            pl.BlockSpec(seq.shape, lambda i: (0, 0)),
            pl.BlockSpec((bm, n), lambda i: (i, 0)),
            pl.BlockSpec(W.shape, lambda i: (0, 0)),
            pl.BlockSpec((1, out_ft), lambda i: (0, 0)),
            pl.BlockSpec((1, 1), lambda i: (0, 0)),
        ],
        out_specs=pl.BlockSpec((bm, out_ft), lambda i: (i, 0)),
        out_shape=jax.ShapeDtypeStruct((n, out_ft), jnp.float32),
        scratch_shapes=[pltpu.VMEM((n, out_ft), jnp.bfloat16)],
    )(seq, adj, W, jnp.reshape(bias, (1, out_ft)),
      jnp.reshape(prelu_a, (1, 1)))


# speedup vs baseline: 1.0165x; 1.0165x over previous
import jax
import jax.numpy as jnp
from jax.experimental import pallas as pl
from jax.experimental.pallas import tpu as pltpu

_BM = 200


def _gcn_kernel(seq_ref, adj_ref, w_ref, bias_ref, a_ref, out_ref, fts_ref):
    @pl.when(pl.program_id(0) == 0)
    def _():
        fts_ref[...] = jax.lax.dot_general(
            seq_ref[...], w_ref[...],
            dimension_numbers=(((1,), (1,)), ((), ())),
            preferred_element_type=jnp.float32)

    out = jax.lax.dot_general(
        adj_ref[...], fts_ref[...],
        dimension_numbers=(((1,), (0,)), ((), ())),
        preferred_element_type=jnp.float32)
    out = out + bias_ref[...]
    a = a_ref[0, 0]
    out_ref[...] = jnp.where(out >= 0, out, a * out)


def kernel(seq, adj, W, bias, prelu_a):
    n, _ = seq.shape
    out_ft = W.shape[0]
    bm = _BM if n % _BM == 0 else min(_BM, n)
    grid = (pl.cdiv(n, bm),)
    return pl.pallas_call(
        _gcn_kernel,
        grid=grid,
        in_specs=[
            pl.BlockSpec(seq.shape, lambda i: (0, 0)),
            pl.BlockSpec((bm, n), lambda i: (i, 0)),
            pl.BlockSpec(W.shape, lambda i: (0, 0)),
            pl.BlockSpec((1, out_ft), lambda i: (0, 0)),
            pl.BlockSpec((1, 1), lambda i: (0, 0)),
        ],
        out_specs=pl.BlockSpec((bm, out_ft), lambda i: (i, 0)),
        out_shape=jax.ShapeDtypeStruct((n, out_ft), jnp.float32),
        scratch_shapes=[pltpu.VMEM((n, out_ft), jnp.float32)],
    )(seq, adj, W, jnp.reshape(bias, (1, out_ft)),
      jnp.reshape(prelu_a, (1, 1)))
